# Initial kernel scaffold; baseline (speedup 1.0000x reference)
import jax
import jax.numpy as jnp
from jax.experimental import pallas as pl


def kernel(xi, xv, ca_emb_weight):
    raise NotImplementedError("write your pallas kernel here")


# speedup vs baseline: 1.2735x; 1.2735x over previous
import functools

import jax
import jax.numpy as jnp
from jax import lax
from jax.experimental import pallas as pl
from jax.experimental.pallas import tpu as pltpu
from jax.experimental.pallas import tpu_sc as plsc

B = 16384
F = 26
EMBD = 32
N = B * F

NC = 2
NS = 16
NW = NC * NS
N_PER_W = N // NW
CHUNK = 128
N_CHUNKS = N_PER_W // CHUNK
LANES = 16


def _scale_rows(rows_v, xv_v):
    @pl.loop(0, CHUNK // LANES)
    def _(g):
        base = g * LANES
        xvv = xv_v[pl.ds(base, LANES)]
        for j in range(LANES):
            s = xvv[j]
            r = base + j
            rows_v.at[r, pl.ds(0, LANES)][...] = (
                rows_v.at[r, pl.ds(0, LANES)][...] * s
            )
            rows_v.at[r, pl.ds(LANES, LANES)][...] = (
                rows_v.at[r, pl.ds(LANES, LANES)][...] * s
            )


def kernel(xi, xv, ca_emb_weight):
    xi_flat = xi.reshape(N).astype(jnp.int32)
    xv_flat = xv.reshape(N)

    mesh = plsc.VectorSubcoreMesh(core_axis_name="c", subcore_axis_name="s")

    @functools.partial(
        pl.kernel,
        out_type=jax.ShapeDtypeStruct((N, EMBD), jnp.float32),
        mesh=mesh,
        scratch_types=[
            pltpu.VMEM((CHUNK,), jnp.int32),
            pltpu.VMEM((CHUNK,), jnp.float32),
            pltpu.VMEM((CHUNK, EMBD), jnp.float32),
            pltpu.SemaphoreType.DMA,
        ],
        compiler_params=pltpu.CompilerParams(use_tc_tiling_on_sc=False),
    )
    def k(table_hbm, idx_hbm, xv_hbm, out_hbm, idx_v, xv_v, rows_v, sem):
        wid = lax.axis_index("s") * NC + lax.axis_index("c")
        base = wid * N_PER_W

        @pl.loop(0, N_CHUNKS)
        def _(ci):
            off = base + ci * CHUNK
            pltpu.sync_copy(idx_hbm.at[pl.ds(off, CHUNK)], idx_v)
            pltpu.sync_copy(xv_hbm.at[pl.ds(off, CHUNK)], xv_v)
            pltpu.async_copy(table_hbm.at[idx_v], rows_v, sem).wait()
            _scale_rows(rows_v, xv_v)
            pltpu.sync_copy(rows_v, out_hbm.at[pl.ds(off, CHUNK)])

    out = k(ca_emb_weight, xi_flat, xv_flat)
    return out.reshape(B, F, EMBD)


# speedup vs baseline: 1.5670x; 1.2305x over previous
import functools

import jax
import jax.numpy as jnp
from jax import lax
from jax.experimental import pallas as pl
from jax.experimental.pallas import tpu as pltpu
from jax.experimental.pallas import tpu_sc as plsc

B = 16384
F = 26
EMBD = 32
N = B * F

NC = 2
NS = 16
NW = NC * NS
CHUNK = 128
N_CHUNKS = N // (NW * CHUNK)
LANES = 16
NBUF = 4


def _scale_rows(src_v, dst_v, xv_v, c):
    @pl.loop(0, CHUNK // LANES)
    def _(g):
        base = g * LANES
        xvv = xv_v[c, pl.ds(base, LANES)]
        for j in range(LANES):
            s = xvv[j]
            r = base + j
            dst_v.at[r, pl.ds(0, LANES)][...] = (
                src_v.at[r, pl.ds(0, LANES)][...] * s
            )
            dst_v.at[r, pl.ds(LANES, LANES)][...] = (
                src_v.at[r, pl.ds(LANES, LANES)][...] * s
            )


def kernel(xi, xv, ca_emb_weight):
    xi_flat = xi.reshape(N // CHUNK, CHUNK).astype(jnp.int32)
    xv_flat = xv.reshape(N // CHUNK, CHUNK)

    mesh = plsc.VectorSubcoreMesh(core_axis_name="c", subcore_axis_name="s")

    @functools.partial(
        pl.kernel,
        out_type=jax.ShapeDtypeStruct((N, EMBD), jnp.float32),
        mesh=mesh,
        scratch_types=[
            pltpu.VMEM((N_CHUNKS, CHUNK), jnp.int32),
            pltpu.VMEM((N_CHUNKS, CHUNK), jnp.float32),
        ]
        + [pltpu.VMEM((CHUNK, EMBD), jnp.float32) for _ in range(2 * NBUF)]
        + [
            pltpu.SemaphoreType.DMA((NBUF,)),
            pltpu.SemaphoreType.DMA((NBUF,)),
        ],
        compiler_params=pltpu.CompilerParams(use_tc_tiling_on_sc=False),
    )
    def k(table_hbm, idx_hbm, xv_hbm, out_hbm, idx_v, xv_v, *bufs_and_sems):
        gbuf = bufs_and_sems[:NBUF]
        obuf = bufs_and_sems[NBUF:2 * NBUF]
        gsem, wsem = bufs_and_sems[2 * NBUF], bufs_and_sems[2 * NBUF + 1]

        wid = lax.axis_index("s") * NC + lax.axis_index("c")
        cbase = wid * N_CHUNKS

        pltpu.sync_copy(idx_hbm.at[pl.ds(cbase, N_CHUNKS)], idx_v)
        pltpu.sync_copy(xv_hbm.at[pl.ds(cbase, N_CHUNKS)], xv_v)

        def start_gather(c, b):
            pltpu.async_copy(table_hbm.at[idx_v.at[c]], gbuf[b], gsem.at[b])

        for b in range(NBUF):
            start_gather(b, b)

        @pl.loop(0, N_CHUNKS, step=NBUF)
        def _(c0):
            for b in range(NBUF):
                c = c0 + b
                pltpu.make_async_copy(
                    table_hbm.at[idx_v.at[c]], gbuf[b], gsem.at[b]
                ).wait()
                @pl.when(c >= NBUF)
                def _():
                    pltpu.make_async_copy(
                        obuf[b],
                        out_hbm.at[pl.ds((cbase + c - NBUF) * CHUNK, CHUNK)],
                        wsem.at[b],
                    ).wait()
                _scale_rows(gbuf[b], obuf[b], xv_v, c)
                pltpu.async_copy(
                    obuf[b],
                    out_hbm.at[pl.ds((cbase + c) * CHUNK, CHUNK)],
                    wsem.at[b],
                )
                @pl.when(c + NBUF < N_CHUNKS)
                def _():
                    start_gather(c + NBUF, b)

        for b in range(NBUF):
            c_tail = N_CHUNKS - NBUF + b
            pltpu.make_async_copy(
                obuf[b],
                out_hbm.at[pl.ds((cbase + c_tail) * CHUNK, CHUNK)],
                wsem.at[b],
            ).wait()

    out = k(ca_emb_weight, xi_flat, xv_flat)
    return out.reshape(B, F, EMBD)
